# baseline (speedup 1.0000x reference)
import functools

import jax
import jax.numpy as jnp
from jax import lax
from jax.experimental import pallas as pl
from jax.experimental.pallas import tpu as pltpu
from jax.experimental.pallas import tpu_sc as plsc

NC = 2
NS = 16
NW = NC * NS

CH = 80


def _sc_aggregate(feature, src, dst, cpt, n_pad):
  n, d = feature.shape
  rpt = n_pad // NS
  assert n_pad % NS == 0 and rpt % CH == 0 and cpt % 2 == 1

  mesh = plsc.VectorSubcoreMesh(core_axis_name="c", subcore_axis_name="s")

  @functools.partial(
      pl.kernel,
      out_type=[
          jax.ShapeDtypeStruct((NC, n_pad, d), jnp.float32),
          jax.ShapeDtypeStruct((NC, n_pad), jnp.float32),
      ],
      mesh=mesh,
      scratch_types=[
          pltpu.VMEM((4, CH, d), jnp.float32),
          pltpu.VMEM((4 * CH,), jnp.int32),
          pltpu.VMEM((4 * CH,), jnp.int32),
          pltpu.VMEM((CH,), jnp.float32),
          pltpu.VMEM((rpt,), jnp.float32),
          pltpu.VMEM_SHARED((n_pad, d), jnp.float32),
          pltpu.VMEM_SHARED((n_pad,), jnp.float32),
          [pltpu.SemaphoreType.DMA] * 4,
          [pltpu.SemaphoreType.DMA] * 4,
      ],
  )
  def agg(feat_hbm, src_hbm, dst_hbm, sums_hbm, degs_hbm,
          rows_v, src_v, dst_v, ones_v, zdeg_v,
          acc_sh, deg_sh, gsem, isem):
    c = lax.axis_index("c")
    s = lax.axis_index("s")
    w = c * NS + s

    zeros16 = jnp.zeros((16,), jnp.float32)
    ones16 = jnp.ones((16,), jnp.float32)

    def fill_rows(r, _):
      for k in range(d // 16):
        rows_v[0, r, pl.ds(k * 16, 16)] = zeros16
      return 0
    lax.fori_loop(0, CH, fill_rows, 0)
    for k in range(CH // 16):
      ones_v[pl.ds(k * 16, 16)] = ones16
    def fill_zdeg(i, _):
      zdeg_v[pl.ds(i * 16, 16)] = zeros16
      return 0
    lax.fori_loop(0, rpt // 16, fill_zdeg, 0)

    for k in range(rpt // CH):
      pltpu.sync_copy(rows_v.at[0], acc_sh.at[pl.ds(s * rpt + k * CH, CH), :])
    pltpu.sync_copy(zdeg_v, deg_sh.at[pl.ds(s * rpt, rpt)])
    plsc.subcore_barrier()

    base = w * cpt * CH

    def fetch(i, q):
      pltpu.async_copy(src_hbm.at[pl.ds(base + i * CH, CH)],
                       src_v.at[pl.ds(q * CH, CH)], isem[q])
      pltpu.async_copy(dst_hbm.at[pl.ds(base + i * CH, CH)],
                       dst_v.at[pl.ds(q * CH, CH)], isem[q])

    def wait_idx(q):
      pltpu.make_async_copy(src_hbm.at[pl.ds(0, CH)],
                            src_v.at[pl.ds(q * CH, CH)], isem[q]).wait()
      pltpu.make_async_copy(dst_hbm.at[pl.ds(0, CH)],
                            dst_v.at[pl.ds(q * CH, CH)], isem[q]).wait()

    def gather(q):
      pltpu.async_copy(feat_hbm.at[src_v.at[pl.ds(q * CH, CH)]],
                       rows_v.at[q], gsem[q])

    def wait_gather(q):
      pltpu.make_async_copy(feat_hbm.at[src_v.at[pl.ds(q * CH, CH)]],
                            rows_v.at[q], gsem[q]).wait()

    def scat(q):
      didx = dst_v.at[pl.ds(q * CH, CH)]
      pltpu.sync_copy(rows_v.at[q], acc_sh.at[didx], add=True)
      pltpu.sync_copy(ones_v, deg_sh.at[didx], add=True)

    for j in range(4):
      fetch(j, j)
    for j in range(3):
      wait_idx(j)
      gather(j)

    nfull = cpt // 4

    def step(p, _):
      for q in range(4):
        i = 4 * p + q
        q3 = (q + 3) % 4
        if q <= 1:
          wait_idx(q3)
          gather(q3)
        else:
          @pl.when(p < nfull - 1)
          def _():
            wait_idx(q3)
            gather(q3)
        wait_gather(q)
        scat(q)
        if q == 0:
          fetch(i + 4, q)
        else:
          @pl.when(p < nfull - 1)
          def _():
            fetch(i + 4, q)
      return 0
    lax.fori_loop(0, nfull, step, 0)

    wait_gather(0)
    scat(0)

    plsc.subcore_barrier()
    pltpu.sync_copy(acc_sh.at[pl.ds(s * rpt, rpt), :],
                    sums_hbm.at[c, pl.ds(s * rpt, rpt), :])
    pltpu.sync_copy(deg_sh.at[pl.ds(s * rpt, rpt)],
                    degs_hbm.at[c, pl.ds(s * rpt, rpt)])

  return agg(feature, src, dst)


def _tc_finish(sums, degt, feature, W, b2d, blk):
  n, d = feature.shape
  d_out = W.shape[1]

  def body(sums_ref, deg_ref, feat_ref, w_ref, b_ref, out_ref):
    i = pl.program_id(0)
    sblk = sums_ref[...]
    ssum = sblk[0] + sblk[1]
    dg = deg_ref[pl.ds(i * blk, blk), :]
    dsum = dg[:, 0:1] + dg[:, 1:2]
    mean = ssum / jnp.maximum(dsum, 1.0)
    h = jnp.where(dsum > 0.0, mean, feat_ref[...])
    acc = jnp.dot(h, w_ref[...], preferred_element_type=jnp.float32)
    out_ref[...] = jnp.maximum(acc + b_ref[...], 0.0)

  return pl.pallas_call(
      body,
      grid=(n // blk,),
      in_specs=[
          pl.BlockSpec((NC, blk, d), lambda i: (0, i, 0)),
          pl.BlockSpec(degt.shape, lambda i: (0, 0)),
          pl.BlockSpec((blk, d), lambda i: (i, 0)),
          pl.BlockSpec((d, d_out), lambda i: (0, 0)),
          pl.BlockSpec((1, d_out), lambda i: (0, 0)),
      ],
      out_specs=pl.BlockSpec((blk, d_out), lambda i: (i, 0)),
      out_shape=jax.ShapeDtypeStruct((n, d_out), jnp.float32),
  )(sums, degt, feature, W, b2d)


def kernel(feature, edge_index, W, b):
  n, d = feature.shape
  e = edge_index.shape[1]
  n_pad = ((n + NS * CH - 1) // (NS * CH)) * (NS * CH)
  cpt = e // (NW * CH)
  sums, degs = _sc_aggregate(feature, edge_index[0], edge_index[1], cpt, n_pad)
  degt = degs.T
  return _tc_finish(sums, degt, feature, W, b.reshape(1, -1), 2000)


# speedup vs baseline: 1.3099x; 1.3099x over previous
import functools

import jax
import jax.numpy as jnp
from jax import lax
from jax.experimental import pallas as pl
from jax.experimental.pallas import tpu as pltpu
from jax.experimental.pallas import tpu_sc as plsc

NC = 2
NS = 16
NW = NC * NS

CH = 80


def _sc_aggregate(feature, src, dst, cpt, n_pad):
  n, d = feature.shape
  rpt = n_pad // NS
  assert n_pad % NS == 0 and rpt % CH == 0 and cpt % 2 == 1

  mesh = plsc.VectorSubcoreMesh(core_axis_name="c", subcore_axis_name="s")

  @functools.partial(
      pl.kernel,
      out_type=[
          jax.ShapeDtypeStruct((NC, n_pad, d), jnp.float32),
          jax.ShapeDtypeStruct((NC, n_pad), jnp.float32),
      ],
      mesh=mesh,
      scratch_types=[
          pltpu.VMEM((4, CH, d), jnp.float32),
          pltpu.VMEM((4 * CH,), jnp.int32),
          pltpu.VMEM((4 * CH,), jnp.int32),
          pltpu.VMEM((CH,), jnp.float32),
          pltpu.VMEM((rpt,), jnp.float32),
          pltpu.VMEM_SHARED((n_pad, d), jnp.float32),
          pltpu.VMEM_SHARED((n_pad,), jnp.float32),
          [pltpu.SemaphoreType.DMA] * 4,
          [pltpu.SemaphoreType.DMA] * 4,
      ],
  )
  def agg(feat_hbm, src_hbm, dst_hbm, sums_hbm, degs_hbm,
          rows_v, src_v, dst_v, ones_v, zdeg_v,
          acc_sh, deg_sh, gsem, isem):
    c = lax.axis_index("c")
    s = lax.axis_index("s")
    w = c * NS + s

    zeros16 = jnp.zeros((16,), jnp.float32)
    ones16 = jnp.ones((16,), jnp.float32)

    def fill_rows(r, _):
      for k in range(d // 16):
        rows_v[0, r, pl.ds(k * 16, 16)] = zeros16
      return 0
    lax.fori_loop(0, CH, fill_rows, 0)
    for k in range(CH // 16):
      ones_v[pl.ds(k * 16, 16)] = ones16
    def fill_zdeg(i, _):
      zdeg_v[pl.ds(i * 16, 16)] = zeros16
      return 0
    lax.fori_loop(0, rpt // 16, fill_zdeg, 0)

    for k in range(rpt // CH):
      pltpu.sync_copy(rows_v.at[0], acc_sh.at[pl.ds(s * rpt + k * CH, CH), :])
    pltpu.sync_copy(zdeg_v, deg_sh.at[pl.ds(s * rpt, rpt)])
    plsc.subcore_barrier()

    base = w * cpt * CH

    def fetch(i, q):
      pltpu.async_copy(src_hbm.at[pl.ds(base + i * CH, CH)],
                       src_v.at[pl.ds(q * CH, CH)], isem[q])
      pltpu.async_copy(dst_hbm.at[pl.ds(base + i * CH, CH)],
                       dst_v.at[pl.ds(q * CH, CH)], isem[q])

    def wait_idx(q):
      pltpu.make_async_copy(src_hbm.at[pl.ds(0, CH)],
                            src_v.at[pl.ds(q * CH, CH)], isem[q]).wait()
      pltpu.make_async_copy(dst_hbm.at[pl.ds(0, CH)],
                            dst_v.at[pl.ds(q * CH, CH)], isem[q]).wait()

    def gather(q):
      pltpu.async_copy(feat_hbm.at[src_v.at[pl.ds(q * CH, CH)]],
                       rows_v.at[q], gsem[q])

    def wait_gather(q):
      pltpu.make_async_copy(feat_hbm.at[src_v.at[pl.ds(q * CH, CH)]],
                            rows_v.at[q], gsem[q]).wait()

    def scat(q):
      didx = dst_v.at[pl.ds(q * CH, CH)]
      pltpu.sync_copy(rows_v.at[q], acc_sh.at[didx], add=True)
      pltpu.sync_copy(ones_v, deg_sh.at[didx], add=True)

    for j in range(4):
      fetch(j, j)
    wait_idx(0)
    gather(0)
    wait_idx(1)
    gather(1)

    nfull = cpt // 4

    def step(p, _):
      for q in range(4):
        i = 4 * p + q
        q2 = (q + 2) % 4
        if q <= 1:
          wait_idx(q2)
          gather(q2)
        else:
          @pl.when(p < nfull - 1)
          def _():
            wait_idx(q2)
            gather(q2)
        wait_gather(q)
        scat(q)
        if q == 0:
          fetch(i + 4, q)
        else:
          @pl.when(p < nfull - 1)
          def _():
            fetch(i + 4, q)
      return 0
    lax.fori_loop(0, nfull, step, 0)

    wait_idx(0)
    gather(0)
    wait_gather(0)
    scat(0)

    plsc.subcore_barrier()
    pltpu.sync_copy(acc_sh.at[pl.ds(s * rpt, rpt), :],
                    sums_hbm.at[c, pl.ds(s * rpt, rpt), :])
    pltpu.sync_copy(deg_sh.at[pl.ds(s * rpt, rpt)],
                    degs_hbm.at[c, pl.ds(s * rpt, rpt)])

  return agg(feature, src, dst)


def _tc_finish(sums, degt, feature, W, b2d, blk):
  n, d = feature.shape
  d_out = W.shape[1]

  def body(sums_ref, deg_ref, feat_ref, w_ref, b_ref, out_ref):
    i = pl.program_id(0)
    sblk = sums_ref[...]
    ssum = sblk[0] + sblk[1]
    dg = deg_ref[pl.ds(i * blk, blk), :]
    dsum = dg[:, 0:1] + dg[:, 1:2]
    mean = ssum / jnp.maximum(dsum, 1.0)
    h = jnp.where(dsum > 0.0, mean, feat_ref[...])
    acc = jnp.dot(h, w_ref[...], preferred_element_type=jnp.float32)
    out_ref[...] = jnp.maximum(acc + b_ref[...], 0.0)

  return pl.pallas_call(
      body,
      grid=(n // blk,),
      in_specs=[
          pl.BlockSpec((NC, blk, d), lambda i: (0, i, 0)),
          pl.BlockSpec(degt.shape, lambda i: (0, 0)),
          pl.BlockSpec((blk, d), lambda i: (i, 0)),
          pl.BlockSpec((d, d_out), lambda i: (0, 0)),
          pl.BlockSpec((1, d_out), lambda i: (0, 0)),
      ],
      out_specs=pl.BlockSpec((blk, d_out), lambda i: (i, 0)),
      out_shape=jax.ShapeDtypeStruct((n, d_out), jnp.float32),
  )(sums, degt, feature, W, b2d)


def kernel(feature, edge_index, W, b):
  n, d = feature.shape
  e = edge_index.shape[1]
  n_pad = ((n + NS * CH - 1) // (NS * CH)) * (NS * CH)
  cpt = e // (NW * CH)
  sums, degs = _sc_aggregate(feature, edge_index[0], edge_index[1], cpt, n_pad)
  degt = degs.T
  return _tc_finish(sums, degt, feature, W, b.reshape(1, -1), 2000)
